# baseline (speedup 1.0000x reference)
import functools

import jax
import jax.numpy as jnp
from jax.experimental import pallas as pl
from jax.experimental.pallas import tpu as pltpu

NSPLIT = 16


def _p_body(x_hbm, o_hbm, sems):
    bs = x_hbm.shape[0]
    bb = bs // NSPLIT
    for i in range(NSPLIT):
        pltpu.make_async_copy(
            x_hbm.at[pl.ds(i * bb, bb)], o_hbm.at[pl.ds(i * bb, bb)],
            sems.at[i]).start()
    for i in range(NSPLIT):
        pltpu.make_async_copy(
            x_hbm.at[pl.ds(i * bb, bb)], o_hbm.at[pl.ds(i * bb, bb)],
            sems.at[i]).wait()


def kernel(x, ts_token_mask, ch_mask, patch_pos_w, ch_pos_w):
    out = pl.pallas_call(
        _p_body,
        in_specs=[pl.BlockSpec(memory_space=pl.ANY)],
        out_specs=pl.BlockSpec(memory_space=pl.ANY),
        out_shape=jax.ShapeDtypeStruct(x.shape, x.dtype),
        scratch_shapes=[pltpu.SemaphoreType.DMA((NSPLIT,))],
    )(x)
    return out


# speedup vs baseline: 6.5637x; 6.5637x over previous
import functools

import jax
import jax.numpy as jnp
from jax import lax
from jax.experimental import pallas as pl
from jax.experimental.pallas import tpu as pltpu
from jax.experimental.pallas import tpu_sc as plsc


def _bias_body(ts_ref, ch_ref, pw_ref, cw_ref, o_ref):
    n_tok = jnp.sum(ts_ref[...])
    n_ch = jnp.sum(ch_ref[...])
    max_n, emb = pw_ref.shape
    max_c = cw_ref.shape[0]
    rows_p = lax.broadcasted_iota(jnp.int32, (max_n, emb), 0)
    sel_p = jnp.where(rows_p < n_tok, pw_ref[...], pw_ref[max_n - 1:max_n, :])
    rows_c = lax.broadcasted_iota(jnp.int32, (max_c, emb), 0)
    sel_c = jnp.where(rows_c < n_ch, cw_ref[...], cw_ref[max_c - 1:max_c, :])
    o_ref[...] = sel_c[:, None, :] + sel_p[None, :, :]


_NC, _NS, _L = 2, 16, 16


def _make_sc_body(bs, max_c, max_n, emb, csplit):
    nw = _NC * _NS
    bpw = bs // nw
    cw = max_c // csplit
    nch = bpw * csplit
    nvec = emb // _L

    def _body(bias_hbm, x_hbm, o_hbm, bias_v, xin, xout, isems, osems):
        wid = lax.axis_index("s") * _NC + lax.axis_index("c")
        base = wid * bpw
        pltpu.sync_copy(bias_hbm, bias_v)

        def in_cp(t, k):
            b = base + t // csplit
            c0 = (t % csplit) * cw
            return pltpu.make_async_copy(
                x_hbm.at[b, pl.ds(c0, cw)], xin.at[k], isems.at[k])

        def out_cp(t, k):
            b = base + t // csplit
            c0 = (t % csplit) * cw
            return pltpu.make_async_copy(
                xout.at[k], o_hbm.at[b, pl.ds(c0, cw)], osems.at[k])

        in_cp(0, 0).start()

        def step(t, carry):
            k = t % 2
            in_cp(t, k).wait()

            @pl.when(t + 1 < nch)
            def _():
                in_cp(t + 1, 1 - k).start()

            @pl.when(t >= 2)
            def _():
                out_cp(t - 2, k).wait()

            c0 = (t % csplit) * cw

            def ci_body(ci, carry2):
                for n in range(max_n):
                    for j in range(nvec):
                        sl = pl.ds(j * _L, _L)
                        xout[k, ci, n, sl] = (
                            xin[k, ci, n, sl] + bias_v[c0 + ci, n, sl])
                return carry2

            lax.fori_loop(0, cw, ci_body, 0)
            out_cp(t, k).start()
            return carry

        lax.fori_loop(0, nch, step, 0)
        out_cp(nch - 2, (nch - 2) % 2).wait()
        out_cp(nch - 1, (nch - 1) % 2).wait()

    return _body


@functools.partial(jax.jit, static_argnames=("csplit",))
def _run(x, ts_i, ch_i, patch_pos_w, ch_pos_w, csplit=3):
    bs, max_c, max_n, emb = x.shape
    bias = pl.pallas_call(
        _bias_body,
        out_shape=jax.ShapeDtypeStruct((max_c, max_n, emb), x.dtype),
    )(ts_i, ch_i, patch_pos_w, ch_pos_w)
    cw = max_c // csplit
    sc_add = functools.partial(
        pl.kernel,
        out_type=jax.ShapeDtypeStruct((bs, max_c, max_n, emb), x.dtype),
        mesh=plsc.VectorSubcoreMesh(core_axis_name="c", subcore_axis_name="s"),
        scratch_types=[
            pltpu.VMEM((max_c, max_n, emb), x.dtype),
            pltpu.VMEM((2, cw, max_n, emb), x.dtype),
            pltpu.VMEM((2, cw, max_n, emb), x.dtype),
            pltpu.SemaphoreType.DMA((2,)),
            pltpu.SemaphoreType.DMA((2,)),
        ],
    )(_make_sc_body(bs, max_c, max_n, emb, csplit))
    return sc_add(bias, x)


def kernel(x, ts_token_mask, ch_mask, patch_pos_w, ch_pos_w):
    ts_i = ts_token_mask.astype(jnp.int32)
    ch_i = ch_mask.astype(jnp.int32)
    return _run(x, ts_i, ch_i, patch_pos_w, ch_pos_w)
